# baseline (speedup 1.0000x reference)
import functools

import jax
import jax.numpy as jnp
from jax import lax
from jax.experimental import pallas as pl
from jax.experimental.pallas import tpu as pltpu
from jax.experimental.pallas import tpu_sc as plsc

NUM_CORES = 2
NUM_SUBCORES = 16
LANES = 16
NUM_WORKERS = NUM_CORES * NUM_SUBCORES
BATCH = 16384
DIM = 64
B_PER_W = BATCH // NUM_WORKERS
CHUNK = 128
NCHUNK = B_PER_W // CHUNK
GROUPS = CHUNK // LANES
PAIR_DIM = 2 * DIM


def _make_kernel():
    mesh = plsc.VectorSubcoreMesh(core_axis_name="c", subcore_axis_name="s")

    @functools.partial(
        pl.kernel,
        mesh=mesh,
        compiler_params=pltpu.CompilerParams(needs_layout_passes=False),
        out_type=jax.ShapeDtypeStruct((BATCH,), jnp.float32),
        scratch_types=[
            pltpu.VMEM((B_PER_W,), jnp.int32),
            pltpu.VMEM((B_PER_W,), jnp.int32),
            pltpu.VMEM((2, CHUNK, PAIR_DIM), jnp.float32),
            pltpu.VMEM((B_PER_W,), jnp.float32),
            pltpu.SemaphoreType.DMA,
        ],
    )
    def probe_kernel(upair_hbm, upar_hbm, ut_hbm, out_hbm,
                     upairv, uparv, ubuf, outv, usem):
        cid = lax.axis_index("c")
        sid = lax.axis_index("s")
        wid = sid * NUM_CORES + cid
        base = wid * B_PER_W

        pltpu.sync_copy(upair_hbm.at[wid], upairv)
        pltpu.sync_copy(upar_hbm.at[wid], uparv)

        pltpu.async_copy(ut_hbm.at[upairv.at[pl.ds(0, CHUNK)]],
                         ubuf.at[0], usem)

        for c in range(NCHUNK):
            slot = c % 2
            pltpu.make_async_copy(ut_hbm.at[upairv.at[pl.ds(c * CHUNK, CHUNK)]],
                                  ubuf.at[slot], usem).wait()
            if c + 1 < NCHUNK:
                nxt = (c + 1) * CHUNK
                pltpu.async_copy(ut_hbm.at[upairv.at[pl.ds(nxt, CHUNK)]],
                                 ubuf.at[1 - slot], usem)

            urows = ubuf.at[slot]

            def group_body(g, carry, urows=urows, c=c):
                rows = g * LANES + lax.iota(jnp.int32, LANES)
                off = c * CHUNK + g * LANES
                ucol = uparv[pl.ds(off, LANES)] * DIM
                u = plsc.load_gather(urows, [rows, ucol])
                outv[pl.ds(off, LANES)] = u
                return carry

            lax.fori_loop(0, GROUPS, group_body, 0)

        pltpu.sync_copy(outv, out_hbm.at[pl.ds(base, B_PER_W)])

    return probe_kernel


_PROBE = _make_kernel()


@jax.jit
def kernel(user, item, user_table, item_table):
    upair = (user >> 1).reshape(NUM_WORKERS, B_PER_W)
    upar = (user & 1).reshape(NUM_WORKERS, B_PER_W)
    ut2 = user_table.reshape(1000000 // 2, PAIR_DIM)
    return _PROBE(upair, upar, ut2)


# speedup vs baseline: 1.4043x; 1.4043x over previous
import functools

import jax
import jax.numpy as jnp
from jax import lax
from jax.experimental import pallas as pl
from jax.experimental.pallas import tpu as pltpu
from jax.experimental.pallas import tpu_sc as plsc

NUM_CORES = 2
NUM_SUBCORES = 16
LANES = 16
NUM_WORKERS = NUM_CORES * NUM_SUBCORES

BATCH = 16384
DIM = 64
B_PER_W = BATCH // NUM_WORKERS
GROUPS = B_PER_W // LANES
WAVE = 2
WAVES = LANES // WAVE
TILE_W = 128


def _make_kernel():
    mesh = plsc.VectorSubcoreMesh(core_axis_name="c", subcore_axis_name="s")

    @functools.partial(
        pl.kernel,
        mesh=mesh,
        compiler_params=pltpu.CompilerParams(needs_layout_passes=False),
        out_type=jax.ShapeDtypeStruct((BATCH,), jnp.float32),
        scratch_types=[
            pltpu.VMEM((B_PER_W,), jnp.int32),
            pltpu.VMEM((B_PER_W,), jnp.int32),
            pltpu.VMEM((2 * WAVE, DIM, TILE_W), jnp.float32),
            pltpu.VMEM((2 * WAVE, DIM, TILE_W), jnp.float32),
            pltpu.VMEM((LANES, LANES), jnp.float32),
            pltpu.VMEM((B_PER_W,), jnp.float32),
            pltpu.SemaphoreType.DMA,
            pltpu.SemaphoreType.DMA,
            pltpu.SemaphoreType.DMA,
            pltpu.SemaphoreType.DMA,
        ],
    )
    def bprmf_kernel(user_hbm, item_hbm, ut_hbm, it_hbm, out_hbm,
                     uidxv, iidxv, ublk, iblk, part, outv,
                     usem0, usem1, isem0, isem1):
        cid = lax.axis_index("c")
        sid = lax.axis_index("s")
        wid = sid * NUM_CORES + cid
        base = wid * B_PER_W

        usems = [usem0, usem1]
        isems = [isem0, isem1]

        pltpu.sync_copy(user_hbm.at[wid], uidxv)
        pltpu.sync_copy(item_hbm.at[wid], iidxv)

        feat = [lax.iota(jnp.int32, LANES) + t * LANES
                for t in range(DIM // LANES)]
        iota16 = lax.iota(jnp.int32, LANES)

        def fire1(r, c, slot, half):
            roff = pl.multiple_of((r >> 7) << 7, TILE_W)
            coff = pl.multiple_of((c >> 7) << 7, TILE_W)
            pltpu.async_copy(ut_hbm.at[:, pl.ds(roff, TILE_W)],
                             ublk.at[slot], usems[half])
            pltpu.async_copy(it_hbm.at[:, pl.ds(coff, TILE_W)],
                             iblk.at[slot], isems[half])

        def fire_wave(ridx, cidx, e0, half):
            for k in range(WAVE):
                fire1(ridx[e0 + k], cidx[e0 + k], half * WAVE + k, half)

        def drain_wave(half):
            for k in range(WAVE):
                pltpu.make_async_copy(
                    ut_hbm.at[:, pl.ds(0, TILE_W)],
                    ublk.at[half * WAVE + k], usems[half]).wait()
                pltpu.make_async_copy(
                    it_hbm.at[:, pl.ds(0, TILE_W)],
                    iblk.at[half * WAVE + k], isems[half]).wait()

        def compute1(r, c, slot, i):
            qu = jnp.full((LANES,), r & 127, jnp.int32)
            qi = jnp.full((LANES,), c & 127, jnp.int32)
            acc = None
            for t in range(DIM // LANES):
                u = plsc.load_gather(ublk.at[slot], [feat[t], qu])
                v = plsc.load_gather(iblk.at[slot], [feat[t], qi])
                uv = u * v
                acc = uv if acc is None else acc + uv
            part[i, pl.ds(0, LANES)] = acc

        ridx0 = uidxv[pl.ds(0, LANES)]
        cidx0 = iidxv[pl.ds(0, LANES)]
        fire_wave(ridx0, cidx0, 0, 0)

        def group_body(g, carry):
            ridx = uidxv[pl.ds(g * LANES, LANES)]
            cidx = iidxv[pl.ds(g * LANES, LANES)]
            gn = jnp.minimum(g + 1, GROUPS - 1) * LANES
            ridxn = uidxv[pl.ds(gn, LANES)]
            cidxn = iidxv[pl.ds(gn, LANES)]

            for w in range(WAVES):
                half = w % 2
                nhalf = (w + 1) % 2
                if w + 1 < WAVES:
                    fire_wave(ridx, cidx, (w + 1) * WAVE, nhalf)
                else:
                    @pl.when(g + 1 < GROUPS)
                    def _():
                        fire_wave(ridxn, cidxn, 0, nhalf)

                drain_wave(half)
                for k in range(WAVE):
                    i = w * WAVE + k
                    compute1(ridx[i], cidx[i], half * WAVE + k, i)

            tot = None
            for l in range(LANES):
                t = plsc.load_gather(part, [iota16, jnp.full((LANES,), l,
                                                             jnp.int32)])
                tot = t if tot is None else tot + t
            outv[pl.ds(g * LANES, LANES)] = tot
            return carry

        lax.fori_loop(0, GROUPS, group_body, 0)

        pltpu.sync_copy(outv, out_hbm.at[pl.ds(base, B_PER_W)])

    return bprmf_kernel


_BPRMF = _make_kernel()


@jax.jit
def kernel(user, item, user_table, item_table):
    user2 = user.reshape(NUM_WORKERS, B_PER_W)
    item2 = item.reshape(NUM_WORKERS, B_PER_W)
    return _BPRMF(user2, item2, user_table.T, item_table.T)


# speedup vs baseline: 1.5630x; 1.1130x over previous
import functools

import jax
import jax.numpy as jnp
from jax import lax
from jax.experimental import pallas as pl
from jax.experimental.pallas import tpu as pltpu
from jax.experimental.pallas import tpu_sc as plsc

NUM_CORES = 2
NUM_SUBCORES = 16
LANES = 16
NUM_WORKERS = NUM_CORES * NUM_SUBCORES

BATCH = 16384
DIM = 64
B_PER_W = BATCH // NUM_WORKERS
GROUPS = B_PER_W // LANES
NSLOT = 4
TILE_W = 128


def _make_kernel():
    mesh = plsc.VectorSubcoreMesh(core_axis_name="c", subcore_axis_name="s")

    @functools.partial(
        pl.kernel,
        mesh=mesh,
        compiler_params=pltpu.CompilerParams(needs_layout_passes=False),
        out_type=jax.ShapeDtypeStruct((BATCH,), jnp.float32),
        scratch_types=[
            pltpu.VMEM((B_PER_W,), jnp.int32),
            pltpu.VMEM((B_PER_W,), jnp.int32),
            pltpu.VMEM((NSLOT, DIM, TILE_W), jnp.float32),
            pltpu.VMEM((NSLOT, DIM, TILE_W), jnp.float32),
            pltpu.VMEM((LANES, LANES), jnp.float32),
            pltpu.VMEM((B_PER_W,), jnp.float32),
            [pltpu.SemaphoreType.DMA] * NSLOT,
            [pltpu.SemaphoreType.DMA] * NSLOT,
        ],
    )
    def bprmf_kernel(user_hbm, item_hbm, ut_hbm, it_hbm, out_hbm,
                     uidxv, iidxv, ublk, iblk, part, outv, usems, isems):
        cid = lax.axis_index("c")
        sid = lax.axis_index("s")
        wid = sid * NUM_CORES + cid
        base = wid * B_PER_W

        pltpu.sync_copy(user_hbm.at[wid], uidxv)
        pltpu.sync_copy(item_hbm.at[wid], iidxv)

        feat = [lax.iota(jnp.int32, LANES) + t * LANES
                for t in range(DIM // LANES)]
        iota16 = lax.iota(jnp.int32, LANES)

        def fire1(r, c, slot):
            roff = pl.multiple_of((r >> 7) << 7, TILE_W)
            coff = pl.multiple_of((c >> 7) << 7, TILE_W)
            pltpu.async_copy(ut_hbm.at[:, pl.ds(roff, TILE_W)],
                             ublk.at[slot], usems[slot])
            pltpu.async_copy(it_hbm.at[:, pl.ds(coff, TILE_W)],
                             iblk.at[slot], isems[slot])

        def drain1(slot):
            pltpu.make_async_copy(ut_hbm.at[:, pl.ds(0, TILE_W)],
                                  ublk.at[slot], usems[slot]).wait()
            pltpu.make_async_copy(it_hbm.at[:, pl.ds(0, TILE_W)],
                                  iblk.at[slot], isems[slot]).wait()

        def compute1(r, c, slot, i):
            qu = jnp.full((LANES,), r & 127, jnp.int32)
            qi = jnp.full((LANES,), c & 127, jnp.int32)
            acc = None
            for t in range(DIM // LANES):
                u = plsc.load_gather(ublk.at[slot], [feat[t], qu])
                v = plsc.load_gather(iblk.at[slot], [feat[t], qi])
                uv = u * v
                acc = uv if acc is None else acc + uv
            part[i, pl.ds(0, LANES)] = acc

        ridx0 = uidxv[pl.ds(0, LANES)]
        cidx0 = iidxv[pl.ds(0, LANES)]
        for i in range(NSLOT):
            fire1(ridx0[i], cidx0[i], i)

        def group_body(g, carry):
            ridx = uidxv[pl.ds(g * LANES, LANES)]
            cidx = iidxv[pl.ds(g * LANES, LANES)]
            gn = jnp.minimum(g + 1, GROUPS - 1) * LANES
            ridxn = uidxv[pl.ds(gn, LANES)]
            cidxn = iidxv[pl.ds(gn, LANES)]

            for i in range(LANES):
                slot = i % NSLOT
                drain1(slot)
                compute1(ridx[i], cidx[i], slot, i)
                if i + NSLOT < LANES:
                    fire1(ridx[i + NSLOT], cidx[i + NSLOT], slot)
                else:
                    j = i + NSLOT - LANES

                    @pl.when(g + 1 < GROUPS)
                    def _(j=j):
                        fire1(ridxn[j], cidxn[j], j)

            tot = None
            for l in range(LANES):
                t = plsc.load_gather(part, [iota16, jnp.full((LANES,), l,
                                                             jnp.int32)])
                tot = t if tot is None else tot + t
            outv[pl.ds(g * LANES, LANES)] = tot
            return carry

        lax.fori_loop(0, GROUPS, group_body, 0)

        pltpu.sync_copy(outv, out_hbm.at[pl.ds(base, B_PER_W)])

    return bprmf_kernel


_BPRMF = _make_kernel()


@jax.jit
def kernel(user, item, user_table, item_table):
    user2 = user.reshape(NUM_WORKERS, B_PER_W)
    item2 = item.reshape(NUM_WORKERS, B_PER_W)
    return _BPRMF(user2, item2, user_table.T, item_table.T)


# speedup vs baseline: 1.5727x; 1.0062x over previous
import functools

import jax
import jax.numpy as jnp
from jax import lax
from jax.experimental import pallas as pl
from jax.experimental.pallas import tpu as pltpu
from jax.experimental.pallas import tpu_sc as plsc

NUM_CORES = 2
NUM_SUBCORES = 16
LANES = 16
NUM_WORKERS = NUM_CORES * NUM_SUBCORES

BATCH = 16384
DIM = 64
B_PER_W = BATCH // NUM_WORKERS
GROUPS = B_PER_W // LANES
NSLOT = 6
PERIOD = 48
NPERIOD = 10
TAIL = B_PER_W - NPERIOD * PERIOD
TILE_W = 128


def _make_kernel():
    mesh = plsc.VectorSubcoreMesh(core_axis_name="c", subcore_axis_name="s")

    @functools.partial(
        pl.kernel,
        mesh=mesh,
        compiler_params=pltpu.CompilerParams(needs_layout_passes=False),
        out_type=jax.ShapeDtypeStruct((BATCH,), jnp.float32),
        scratch_types=[
            pltpu.VMEM((B_PER_W,), jnp.int32),
            pltpu.VMEM((B_PER_W,), jnp.int32),
            pltpu.VMEM((NSLOT, DIM, TILE_W), jnp.float32),
            pltpu.VMEM((NSLOT, DIM, TILE_W), jnp.float32),
            pltpu.VMEM((LANES, LANES), jnp.float32),
            pltpu.VMEM((B_PER_W,), jnp.float32),
            [pltpu.SemaphoreType.DMA] * NSLOT,
            [pltpu.SemaphoreType.DMA] * NSLOT,
        ],
    )
    def bprmf_kernel(user_hbm, item_hbm, ut_hbm, it_hbm, out_hbm,
                     uidxv, iidxv, ublk, iblk, part, outv, usems, isems):
        cid = lax.axis_index("c")
        sid = lax.axis_index("s")
        wid = sid * NUM_CORES + cid
        base = wid * B_PER_W

        pltpu.sync_copy(user_hbm.at[wid], uidxv)
        pltpu.sync_copy(item_hbm.at[wid], iidxv)

        feat = [lax.iota(jnp.int32, LANES) + t * LANES
                for t in range(DIM // LANES)]
        iota16 = lax.iota(jnp.int32, LANES)

        def fire1(r, c, slot):
            roff = pl.multiple_of((r >> 7) << 7, TILE_W)
            coff = pl.multiple_of((c >> 7) << 7, TILE_W)
            pltpu.async_copy(ut_hbm.at[:, pl.ds(roff, TILE_W)],
                             ublk.at[slot], usems[slot])
            pltpu.async_copy(it_hbm.at[:, pl.ds(coff, TILE_W)],
                             iblk.at[slot], isems[slot])

        def drain1(slot):
            pltpu.make_async_copy(ut_hbm.at[:, pl.ds(0, TILE_W)],
                                  ublk.at[slot], usems[slot]).wait()
            pltpu.make_async_copy(it_hbm.at[:, pl.ds(0, TILE_W)],
                                  iblk.at[slot], isems[slot]).wait()

        def compute1(r, c, slot, i):
            qu = jnp.full((LANES,), r & 127, jnp.int32)
            qi = jnp.full((LANES,), c & 127, jnp.int32)
            acc = None
            for t in range(DIM // LANES):
                u = plsc.load_gather(ublk.at[slot], [feat[t], qu])
                v = plsc.load_gather(iblk.at[slot], [feat[t], qi])
                uv = u * v
                acc = uv if acc is None else acc + uv
            part[i, pl.ds(0, LANES)] = acc

        def transpose_out(goff):
            tot = None
            for l in range(LANES):
                t = plsc.load_gather(part, [iota16, jnp.full((LANES,), l,
                                                             jnp.int32)])
                tot = t if tot is None else tot + t
            outv[pl.ds(goff, LANES)] = tot

        ridx0 = uidxv[pl.ds(0, LANES)]
        cidx0 = iidxv[pl.ds(0, LANES)]
        for i in range(NSLOT):
            fire1(ridx0[i], cidx0[i], i)

        def period_body(p, carry):
            g0 = 3 * p
            rg = [uidxv[pl.ds((g0 + t) * LANES, LANES)] for t in range(4)]
            cg = [iidxv[pl.ds((g0 + t) * LANES, LANES)] for t in range(4)]

            for el in range(PERIOD):
                slot = el % NSLOT
                gg, i = divmod(el, LANES)
                drain1(slot)
                compute1(rg[gg][i], cg[gg][i], slot, i)
                tg, tl = divmod(el + NSLOT, LANES)
                fire1(rg[tg][tl], cg[tg][tl], (el + NSLOT) % NSLOT)
                if i == LANES - 1:
                    transpose_out((g0 + gg) * LANES)
            return carry

        lax.fori_loop(0, NPERIOD, period_body, 0)

        E0 = NPERIOD * PERIOD
        rgt = [uidxv[pl.ds((E0 // LANES + t) * LANES, LANES)]
               for t in range(TAIL // LANES)]
        cgt = [iidxv[pl.ds((E0 // LANES + t) * LANES, LANES)]
               for t in range(TAIL // LANES)]
        for el in range(TAIL):
            slot = el % NSLOT
            gg, i = divmod(el, LANES)
            drain1(slot)
            compute1(rgt[gg][i], cgt[gg][i], slot, i)
            if el + NSLOT < TAIL:
                tg, tl = divmod(el + NSLOT, LANES)
                fire1(rgt[tg][tl], cgt[tg][tl], (el + NSLOT) % NSLOT)
            if i == LANES - 1:
                transpose_out(E0 + gg * LANES)

        pltpu.sync_copy(outv, out_hbm.at[pl.ds(base, B_PER_W)])

    return bprmf_kernel


_BPRMF = _make_kernel()


@jax.jit
def kernel(user, item, user_table, item_table):
    user2 = user.reshape(NUM_WORKERS, B_PER_W)
    item2 = item.reshape(NUM_WORKERS, B_PER_W)
    return _BPRMF(user2, item2, user_table.T, item_table.T)
